# Initial kernel scaffold; baseline (speedup 1.0000x reference)
import jax
import jax.numpy as jnp
from jax.experimental import pallas as pl


def kernel(ophits_x, pmt_x, flash_x, evt_x, params, e1, e2, e3):
    raise NotImplementedError("write your pallas kernel here")


# speedup vs baseline: 1.6056x; 1.6056x over previous
import functools

import jax
import jax.numpy as jnp
from jax import lax
from jax.experimental import pallas as pl
from jax.experimental.pallas import tpu as pltpu
from jax.experimental.pallas import tpu_sc as plsc

D = 256
NSC = 2
NTILE = 16
NW = NSC * NTILE


def _pre_body(x_ref, w_ref, a_ref, m_ref):
    i = pl.program_id(0)
    x = x_ref[...]
    a_ref[...] = jnp.dot(x, w_ref[...], preferred_element_type=jnp.float32)
    part = jnp.max(jnp.abs(x), axis=0, keepdims=True)

    @pl.when(i == 0)
    def _():
        m_ref[...] = part

    @pl.when(i > 0)
    def _():
        m_ref[...] = jnp.maximum(m_ref[...], part)


def _pre(x, wcol):
    n = x.shape[0]
    trp = n if n <= 1000 else 1000
    assert n % trp == 0
    a, m = pl.pallas_call(
        _pre_body,
        grid=(n // trp,),
        in_specs=[
            pl.BlockSpec((trp, D), lambda i: (i, 0)),
            pl.BlockSpec((D, 1), lambda i: (0, 0)),
        ],
        out_specs=[
            pl.BlockSpec((trp, 1), lambda i: (i, 0)),
            pl.BlockSpec((1, D), lambda i: (0, 0)),
        ],
        out_shape=[
            jax.ShapeDtypeStruct((n, 1), jnp.float32),
            jax.ShapeDtypeStruct((1, D), jnp.float32),
        ],
    )(x, wcol)
    return a[:, 0], m[0]


def _mish(x):
    sp = jnp.maximum(x, 0.0) + jnp.log1p(jnp.exp(-jnp.abs(x)))
    return x * jnp.tanh(sp)


def _mlp_body(ag_ref, xt_ref, w1a_ref, w1b_ref, b1_ref, w2_ref, b2_ref, o_ref):
    h = jnp.dot(ag_ref[...], w1a_ref[...], preferred_element_type=jnp.float32)
    h = h + jnp.dot(xt_ref[...], w1b_ref[...], preferred_element_type=jnp.float32)
    h = _mish(h + b1_ref[...])
    h2 = jnp.dot(h, w2_ref[...], preferred_element_type=jnp.float32) + b2_ref[...]
    o_ref[...] = _mish(h2)


def _mlp(aggr, x_tgt, w1, b1, w2, b2):
    n = aggr.shape[0]
    tr = n if n <= 1000 else 1000
    assert n % tr == 0
    w1a, w1b = w1[:D], w1[D:]
    return pl.pallas_call(
        _mlp_body,
        grid=(n // tr,),
        in_specs=[
            pl.BlockSpec((tr, D), lambda i: (i, 0)),
            pl.BlockSpec((tr, D), lambda i: (i, 0)),
            pl.BlockSpec((D, D), lambda i: (0, 0)),
            pl.BlockSpec((D, D), lambda i: (0, 0)),
            pl.BlockSpec((1, D), lambda i: (0, 0)),
            pl.BlockSpec((D, D), lambda i: (0, 0)),
            pl.BlockSpec((1, D), lambda i: (0, 0)),
        ],
        out_specs=pl.BlockSpec((tr, D), lambda i: (i, 0)),
        out_shape=jax.ShapeDtypeStruct((n, D), jnp.float32),
    )(aggr, x_tgt, w1a, w1b, b1.reshape(1, D), w2, b2.reshape(1, D))


def _ceil(a, b):
    return -(-a // b)


def _sc_cfg(num_tgt, num_edges):
    epad = _ceil(num_edges, 2048) * 2048
    ne = epad // NTILE
    per_tile_words = 2 * ne + 34000
    budget = 2097151 - 32768 - NTILE * per_tile_words
    r_cap = (budget // (4 * 128 + NTILE) - 16) // 128 * 128
    half = _ceil(num_tgt, 2)
    rounds = _ceil(half, r_cap)
    r = _ceil(_ceil(half, rounds), 128) * 128
    tpad = NSC * r * rounds
    return epad, ne, r, rounds, tpad


def _sc_block(S, T, E):
    epad, ne, R, rounds, tpad = _sc_cfg(T, E)
    nv = ne // 16
    rt = R // NTILE
    assert rt % 8 == 0

    mesh = plsc.VectorSubcoreMesh(core_axis_name="c", subcore_axis_name="s")

    def body(xsrc, srcp, dstp, aip, ajp, mvec, consts, out,
             eidx_s, eidx_d, wl_s, wl_d, m_v, c_v, aibuf, ajb, wbuf,
             gbuf, cb0, cb1, cb2, cb3, rb0, rb1, rb2, rb3,
             obuf, zbuf, ac0, ac1, ac2, ac3, sem):
        cid = lax.axis_index("c")
        sid = lax.axis_index("s")

        pltpu.sync_copy(srcp.at[pl.ds(sid * ne, ne)], eidx_s)
        pltpu.sync_copy(dstp.at[pl.ds(sid * ne, ne)], eidx_d)
        pltpu.sync_copy(mvec, m_v)
        pltpu.sync_copy(consts, c_v)
        cv = c_v[...]
        be = cv[0]
        tt = cv[1]
        ta = cv[2]

        for j in range(16):
            m_v[pl.ds(j * 16, 16)] = m_v[pl.ds(j * 16, 16)] * ta

        for zr in range(8):
            for j in range(8):
                zbuf[zr, pl.ds(j * 16, 16)] = jnp.zeros((16,), jnp.float32)

        def round_body(rnd, _):
            lo = (cid * rounds + rnd) * R

            pltpu.sync_copy(aip.at[pl.ds(lo, R)], aibuf.at[pl.ds(0, R)])
            wbuf[pl.ds(0, 16)] = jnp.zeros((16,), jnp.float32)
            aibuf[pl.ds(R, 16)] = jnp.zeros((16,), jnp.float32)

            zds = []
            for zi in range(rt // 8):
                zsl = pl.ds(sid * rt + zi * 8, 8)
                zds.append(pltpu.async_copy(zbuf, ac0.at[zsl], sem))
                zds.append(pltpu.async_copy(zbuf, ac1.at[zsl], sem))
                zds.append(pltpu.async_copy(zbuf, ac2.at[zsl], sem))
                zds.append(pltpu.async_copy(zbuf, ac3.at[zsl], sem))
            for zd in zds:
                zd.wait()
            plsc.subcore_barrier()

            def process_batch(row):
                dg1 = pltpu.async_copy(ajp.at[wl_s.at[row]], ajb, sem)
                dg2 = pltpu.async_copy(xsrc.at[wl_s.at[row]], gbuf, sem)
                dg1.wait()
                dg2.wait()
                for h in range(2):
                    hsl = pl.ds(h * 16, 16)
                    dloc = wl_d[row, hsl]
                    aiv = plsc.load_gather(aibuf, [dloc])
                    z = aiv + ajb[hsl] + be
                    wbuf[hsl] = 1.0 / (1.0 + jnp.exp(-z))

                def edge_body(k, _3):
                    wk = plsc.load_gather(wbuf, [jnp.full((16,), k, jnp.int32)])
                    for j in range(16):
                        sl = pl.ds(j * 16, 16)
                        hsl = pl.ds((j % 8) * 16, 16)
                        xj = gbuf[k, sl]
                        msgv = wk * xj
                        sv = tt * msgv
                        ev = jnp.exp(sv - m_v[sl])
                        cnum = cb0 if j < 8 else cb1
                        cden = cb2 if j < 8 else cb3
                        cnum[k, hsl] = ev * msgv
                        cden[k, hsl] = ev
                    return 0
                lax.fori_loop(0, 32, edge_body, 0)

                da0 = pltpu.async_copy(cb0, ac0.at[wl_d.at[row]], add=True, sem=sem)
                da1 = pltpu.async_copy(cb1, ac1.at[wl_d.at[row]], add=True, sem=sem)
                da2 = pltpu.async_copy(cb2, ac2.at[wl_d.at[row]], add=True, sem=sem)
                da3 = pltpu.async_copy(cb3, ac3.at[wl_d.at[row]], add=True, sem=sem)
                da0.wait()
                da1.wait()
                da2.wait()
                da3.wait()

            def wl_body(v, carry):
                cnt, done = carry
                sl = pl.ds(v * 16, 16)
                dv = eidx_d[sl]
                msk = (dv >= lo) & (dv < lo + R)
                pos = plsc.cumsum(msk.astype(jnp.int32))
                idx = cnt + pos - 1
                irow = lax.bitwise_and(lax.shift_right_logical(idx, 5), 3)
                icol = lax.bitwise_and(idx, 31)
                plsc.store_scatter(wl_d, [irow, icol], dv - lo, mask=msk)
                plsc.store_scatter(wl_s, [irow, icol], eidx_s[sl], mask=msk)
                cnt = cnt + pos[15]
                pend = cnt - done

                @pl.when(pend >= 32)
                def _():
                    process_batch(lax.bitwise_and(
                        lax.shift_right_logical(done, 5), 3))
                done = done + jnp.where(pend >= 32, 32, 0)
                return (cnt, done)
            cnt, done = lax.fori_loop(
                0, nv, wl_body, (jnp.int32(0), jnp.int32(0)))

            for h in range(2):
                pp = cnt + lax.iota(jnp.int32, 16) + 16 * h
                prow = lax.bitwise_and(lax.shift_right_logical(pp, 5), 3)
                pcol = lax.bitwise_and(pp, 31)
                plsc.store_scatter(wl_d, [prow, pcol],
                                   jnp.full((16,), R, jnp.int32))
                plsc.store_scatter(wl_s, [prow, pcol],
                                   jnp.zeros((16,), jnp.int32))
            for h in range(2):
                dcur = done + 32 * h

                @pl.when(dcur < cnt)
                def _():
                    process_batch(lax.bitwise_and(
                        lax.shift_right_logical(dcur, 5), 3))
            plsc.subcore_barrier()

            base = sid * rt

            def ro_body(ri, _4):
                rsl = pl.ds(base + ri * 8, 8)
                rd0 = pltpu.async_copy(ac0.at[rsl], rb0, sem)
                rd1 = pltpu.async_copy(ac1.at[rsl], rb1, sem)
                rd2 = pltpu.async_copy(ac2.at[rsl], rb2, sem)
                rd3 = pltpu.async_copy(ac3.at[rsl], rb3, sem)
                rd0.wait()
                rd1.wait()
                rd2.wait()
                rd3.wait()
                for rr in range(8):
                    for j in range(16):
                        hsl = pl.ds((j % 8) * 16, 16)
                        num = (rb0 if j < 8 else rb1)[rr, hsl]
                        den = (rb2 if j < 8 else rb3)[rr, hsl]
                        obuf[rr, pl.ds(j * 16, 16)] = num / (den + 1e-16)
                pltpu.sync_copy(obuf, out.at[pl.ds(lo + base + ri * 8, 8)])
                return 0
            lax.fori_loop(0, rt // 8, ro_body, 0)
            plsc.subcore_barrier()
            return 0

        lax.fori_loop(0, rounds, round_body, 0)

    run = functools.partial(
        pl.kernel,
        mesh=mesh,
        out_type=jax.ShapeDtypeStruct((tpad, D), jnp.float32),
        compiler_params=pltpu.CompilerParams(needs_layout_passes=False),
        scratch_types=[
            pltpu.VMEM((ne,), jnp.int32),
            pltpu.VMEM((ne,), jnp.int32),
            pltpu.VMEM((4, 32), jnp.int32),
            pltpu.VMEM((4, 32), jnp.int32),
            pltpu.VMEM((D,), jnp.float32),
            pltpu.VMEM((16,), jnp.float32),
            pltpu.VMEM((R + 16,), jnp.float32),
            pltpu.VMEM((32,), jnp.float32),
            pltpu.VMEM((32,), jnp.float32),
            pltpu.VMEM((32, D), jnp.float32),
            pltpu.VMEM((32, 128), jnp.float32),
            pltpu.VMEM((32, 128), jnp.float32),
            pltpu.VMEM((32, 128), jnp.float32),
            pltpu.VMEM((32, 128), jnp.float32),
            pltpu.VMEM((8, 128), jnp.float32),
            pltpu.VMEM((8, 128), jnp.float32),
            pltpu.VMEM((8, 128), jnp.float32),
            pltpu.VMEM((8, 128), jnp.float32),
            pltpu.VMEM((8, D), jnp.float32),
            pltpu.VMEM((8, 128), jnp.float32),
            pltpu.VMEM_SHARED((R + 16, 128), jnp.float32),
            pltpu.VMEM_SHARED((R + 16, 128), jnp.float32),
            pltpu.VMEM_SHARED((R + 16, 128), jnp.float32),
            pltpu.VMEM_SHARED((R + 16, 128), jnp.float32),
            pltpu.SemaphoreType.DMA,
        ],
    )(body)
    return run, epad, tpad


def _sc_aggregate(p, x_src, src, dst, num_tgt, a_i, a_j, m):
    S, E = x_src.shape[0], src.shape[0]
    run, epad, tpad = _sc_block(S, num_tgt, E)
    srcp = jnp.pad(src, (0, epad - E))
    dstp = jnp.pad(dst, (0, epad - E), constant_values=tpad)
    aip = jnp.pad(a_i, (0, tpad - num_tgt))
    consts = jnp.concatenate([
        p["be"].reshape(1), p["t"].reshape(1), jnp.abs(p["t"]).reshape(1),
        jnp.zeros((13,), jnp.float32)])
    out = run(x_src, srcp, dstp, aip, a_j, m, consts)
    return out[:num_tgt]


def _block(p, x_src, x_tgt, src, dst, num_tgt):
    a_i, _ = _pre(x_tgt, p["We"][:D])
    a_j, m = _pre(x_src, p["We"][D:])
    aggr = _sc_aggregate(p, x_src, src, dst, num_tgt, a_i, a_j, m)
    return _mlp(aggr, x_tgt, p["W1"], p["b1"], p["W2"], p["b2"])


def kernel(ophits_x, pmt_x, flash_x, evt_x, params, e1, e2, e3):
    n_ophit, n_pmt = ophits_x.shape[0], pmt_x.shape[0]
    n_flash, n_evt = flash_x.shape[0], evt_x.shape[0]
    pmt = _block(params["ophit_to_pmt"], ophits_x, pmt_x, e1[0], e1[1], n_pmt)
    flash = _block(params["pmt_to_flash"], pmt, flash_x, e2[0], e2[1], n_flash)
    evt = _block(params["flash_to_interaction"], flash, evt_x, e3[0], e3[1], n_evt)
    flash = _block(params["interaction_to_flash"], evt, flash, e3[1], e3[0], n_flash)
    pmt = _block(params["flash_to_pmt"], flash, pmt, e2[1], e2[0], n_pmt)
    ophits = _block(params["pmt_to_ophit"], pmt, ophits_x, e1[1], e1[0], n_ophit)
    return (ophits, pmt, flash, evt)
